# baseline (speedup 1.0000x reference)
import functools

import jax
import jax.numpy as jnp
from jax import lax
from jax.experimental import pallas as pl
from jax.experimental.pallas import tpu as pltpu
from jax.experimental.pallas import tpu_sc as plsc

MIN_VAL = -8.0
MAX_VAL = 8.0
SCALE = (MAX_VAL - MIN_VAL) / 15.0
MAGIC = 12582912.0

ROWS = 4096
COLS = 16384
TC_ROWS = 2560
SC_ROWS = ROWS - TC_ROWS
NWORKERS = 32
SC_ROWS_PER_WORKER = SC_ROWS // NWORKERS
LANES = 16
UNROLL = 16

CHUNK = COLS // 2
CHUNKS_PER_WORKER = SC_ROWS_PER_WORKER * 2
NBUF = 4

TC_BLOCK = 128


def _quant_vec(x):
    t = jnp.minimum(jnp.maximum(x, MIN_VAL), MAX_VAL)
    u = (t + (MAGIC + 8.0)) - MAGIC
    b = jnp.where(u >= 8.0, MIN_VAL - SCALE, MIN_VAL)
    return u * SCALE + b


def _quantize_chunk(src, dst):
    @plsc.parallel_loop(0, CHUNK, step=LANES, unroll=UNROLL)
    def vbody(i):
        sl = pl.ds(i, LANES)
        dst[sl] = _quant_vec(src[sl])


def _sc_body(x_hbm, out_hbm, in_bufs, out_bufs, in_sems, out_sems):
    wid = lax.axis_index("s") * 2 + lax.axis_index("c")
    base = TC_ROWS + wid * SC_ROWS_PER_WORKER

    def chunk_slice(k):
        row = base + lax.div(k, 2)
        col = lax.rem(k, 2) * CHUNK
        return (row, pl.ds(col, CHUNK))

    for b in range(NBUF):
        pltpu.async_copy(x_hbm.at[chunk_slice(jnp.int32(b))], in_bufs[b], in_sems[b])

    steps = CHUNKS_PER_WORKER // NBUF

    def g_body(g, carry):
        for b in range(NBUF):
            k = g * NBUF + b

            @pl.when(g > 0)
            def _():
                pltpu.make_async_copy(
                    out_bufs[b], out_hbm.at[chunk_slice(k)], out_sems[b]
                ).wait()

            pltpu.make_async_copy(
                x_hbm.at[chunk_slice(k)], in_bufs[b], in_sems[b]
            ).wait()
            _quantize_chunk(in_bufs[b], out_bufs[b])
            pltpu.async_copy(out_bufs[b], out_hbm.at[chunk_slice(k)], out_sems[b])

            @pl.when(g < steps - 1)
            def _():
                pltpu.async_copy(
                    x_hbm.at[chunk_slice(k + NBUF)], in_bufs[b], in_sems[b]
                )

        return carry

    lax.fori_loop(0, steps, g_body, 0)

    for b in range(NBUF):
        pltpu.make_async_copy(
            out_bufs[b], out_hbm.at[chunk_slice(jnp.int32(b))], out_sems[b]
        ).wait()


@functools.partial(
    pl.kernel,
    out_type=jax.ShapeDtypeStruct((ROWS, COLS), jnp.float32),
    mesh=plsc.VectorSubcoreMesh(core_axis_name="c", subcore_axis_name="s"),
    scratch_types=[
        [pltpu.VMEM((CHUNK,), jnp.float32)] * NBUF,
        [pltpu.VMEM((CHUNK,), jnp.float32)] * NBUF,
        [pltpu.SemaphoreType.DMA] * NBUF,
        [pltpu.SemaphoreType.DMA] * NBUF,
    ],
)
def _sc_quantize_bottom(x_hbm, out_hbm, in_bufs, out_bufs, in_sems, out_sems):
    _sc_body(x_hbm, out_hbm, in_bufs, out_bufs, in_sems, out_sems)


def _tc_quant_body(x_ref, partial_hbm_ref, o_ref):
    o_ref[...] = _quant_vec(x_ref[...])


def _tc_quantize_top_inplace(x, sc_partial):
    return pl.pallas_call(
        _tc_quant_body,
        grid=(TC_ROWS // TC_BLOCK,),
        in_specs=[
            pl.BlockSpec((TC_BLOCK, COLS), lambda i: (i, 0)),
            pl.BlockSpec(memory_space=pl.ANY),
        ],
        out_specs=pl.BlockSpec((TC_BLOCK, COLS), lambda i: (i, 0)),
        out_shape=jax.ShapeDtypeStruct((ROWS, COLS), jnp.float32),
        input_output_aliases={1: 0},
    )(x, sc_partial)


def kernel(x):
    sc_partial = _sc_quantize_bottom(x)
    return _tc_quantize_top_inplace(x, sc_partial)


# speedup vs baseline: 1.0248x; 1.0248x over previous
import functools

import jax
import jax.numpy as jnp
from jax import lax
from jax.experimental import pallas as pl
from jax.experimental.pallas import tpu as pltpu
from jax.experimental.pallas import tpu_sc as plsc

MIN_VAL = -8.0
MAX_VAL = 8.0
SCALE = (MAX_VAL - MIN_VAL) / 15.0
MAGIC = 12582912.0

ROWS = 4096
COLS = 16384
TC_ROWS = 3072
SC_ROWS = ROWS - TC_ROWS
NWORKERS = 32
SC_ROWS_PER_WORKER = SC_ROWS // NWORKERS
LANES = 16
UNROLL = 16

CHUNK = COLS // 2
CHUNKS_PER_WORKER = SC_ROWS_PER_WORKER * 2
NBUF = 4

TC_BLOCK = 128


def _quant_vec(x):
    t = jnp.minimum(jnp.maximum(x, MIN_VAL), MAX_VAL)
    u = (t + (MAGIC + 8.0)) - MAGIC
    b = jnp.where(u >= 8.0, MIN_VAL - SCALE, MIN_VAL)
    return u * SCALE + b


def _quantize_chunk(src, dst):
    @plsc.parallel_loop(0, CHUNK, step=LANES, unroll=UNROLL)
    def vbody(i):
        sl = pl.ds(i, LANES)
        dst[sl] = _quant_vec(src[sl])


def _sc_body(x_hbm, out_hbm, in_bufs, out_bufs, in_sems, out_sems):
    wid = lax.axis_index("s") * 2 + lax.axis_index("c")
    base = TC_ROWS + wid * SC_ROWS_PER_WORKER

    def chunk_slice(k):
        row = base + lax.div(k, 2)
        col = lax.rem(k, 2) * CHUNK
        return (row, pl.ds(col, CHUNK))

    for b in range(NBUF):
        pltpu.async_copy(x_hbm.at[chunk_slice(jnp.int32(b))], in_bufs[b], in_sems[b])

    steps = CHUNKS_PER_WORKER // NBUF

    def g_body(g, carry):
        for b in range(NBUF):
            k = g * NBUF + b

            @pl.when(g > 0)
            def _():
                pltpu.make_async_copy(
                    out_bufs[b], out_hbm.at[chunk_slice(k)], out_sems[b]
                ).wait()

            pltpu.make_async_copy(
                x_hbm.at[chunk_slice(k)], in_bufs[b], in_sems[b]
            ).wait()
            _quantize_chunk(in_bufs[b], out_bufs[b])
            pltpu.async_copy(out_bufs[b], out_hbm.at[chunk_slice(k)], out_sems[b])

            @pl.when(g < steps - 1)
            def _():
                pltpu.async_copy(
                    x_hbm.at[chunk_slice(k + NBUF)], in_bufs[b], in_sems[b]
                )

        return carry

    lax.fori_loop(0, steps, g_body, 0)

    for b in range(NBUF):
        pltpu.make_async_copy(
            out_bufs[b], out_hbm.at[chunk_slice(jnp.int32(b))], out_sems[b]
        ).wait()


@functools.partial(
    pl.kernel,
    out_type=jax.ShapeDtypeStruct((ROWS, COLS), jnp.float32),
    mesh=plsc.VectorSubcoreMesh(core_axis_name="c", subcore_axis_name="s"),
    scratch_types=[
        [pltpu.VMEM((CHUNK,), jnp.float32)] * NBUF,
        [pltpu.VMEM((CHUNK,), jnp.float32)] * NBUF,
        [pltpu.SemaphoreType.DMA] * NBUF,
        [pltpu.SemaphoreType.DMA] * NBUF,
    ],
)
def _sc_quantize_bottom(x_hbm, out_hbm, in_bufs, out_bufs, in_sems, out_sems):
    _sc_body(x_hbm, out_hbm, in_bufs, out_bufs, in_sems, out_sems)


def _tc_quant_body(x_ref, partial_hbm_ref, o_ref):
    o_ref[...] = _quant_vec(x_ref[...])


def _tc_quantize_top_inplace(x, sc_partial):
    return pl.pallas_call(
        _tc_quant_body,
        grid=(TC_ROWS // TC_BLOCK,),
        in_specs=[
            pl.BlockSpec((TC_BLOCK, COLS), lambda i: (i, 0)),
            pl.BlockSpec(memory_space=pl.ANY),
        ],
        out_specs=pl.BlockSpec((TC_BLOCK, COLS), lambda i: (i, 0)),
        out_shape=jax.ShapeDtypeStruct((ROWS, COLS), jnp.float32),
        input_output_aliases={1: 0},
    )(x, sc_partial)


def kernel(x):
    sc_partial = _sc_quantize_bottom(x)
    return _tc_quantize_top_inplace(x, sc_partial)


# speedup vs baseline: 1.0523x; 1.0269x over previous
import functools

import jax
import jax.numpy as jnp
from jax import lax
from jax.experimental import pallas as pl
from jax.experimental.pallas import tpu as pltpu
from jax.experimental.pallas import tpu_sc as plsc

MIN_VAL = -8.0
MAX_VAL = 8.0
SCALE = (MAX_VAL - MIN_VAL) / 15.0
MAGIC = 12582912.0

ROWS = 4096
COLS = 16384
TC_ROWS = 3584
SC_ROWS = ROWS - TC_ROWS
NWORKERS = 32
SC_ROWS_PER_WORKER = SC_ROWS // NWORKERS
LANES = 16
UNROLL = 16

CHUNK = COLS // 2
CHUNKS_PER_WORKER = SC_ROWS_PER_WORKER * 2
NBUF = 4

TC_BLOCK = 128


def _quant_vec(x):
    t = jnp.minimum(jnp.maximum(x, MIN_VAL), MAX_VAL)
    u = (t + (MAGIC + 8.0)) - MAGIC
    b = jnp.where(u >= 8.0, MIN_VAL - SCALE, MIN_VAL)
    return u * SCALE + b


def _quantize_chunk(src, dst):
    @plsc.parallel_loop(0, CHUNK, step=LANES, unroll=UNROLL)
    def vbody(i):
        sl = pl.ds(i, LANES)
        dst[sl] = _quant_vec(src[sl])


def _sc_body(x_hbm, out_hbm, in_bufs, out_bufs, in_sems, out_sems):
    wid = lax.axis_index("s") * 2 + lax.axis_index("c")
    base = TC_ROWS + wid * SC_ROWS_PER_WORKER

    def chunk_slice(k):
        row = base + lax.div(k, 2)
        col = lax.rem(k, 2) * CHUNK
        return (row, pl.ds(col, CHUNK))

    for b in range(NBUF):
        pltpu.async_copy(x_hbm.at[chunk_slice(jnp.int32(b))], in_bufs[b], in_sems[b])

    steps = CHUNKS_PER_WORKER // NBUF

    def g_body(g, carry):
        for b in range(NBUF):
            k = g * NBUF + b

            @pl.when(g > 0)
            def _():
                pltpu.make_async_copy(
                    out_bufs[b], out_hbm.at[chunk_slice(k)], out_sems[b]
                ).wait()

            pltpu.make_async_copy(
                x_hbm.at[chunk_slice(k)], in_bufs[b], in_sems[b]
            ).wait()
            _quantize_chunk(in_bufs[b], out_bufs[b])
            pltpu.async_copy(out_bufs[b], out_hbm.at[chunk_slice(k)], out_sems[b])

            @pl.when(g < steps - 1)
            def _():
                pltpu.async_copy(
                    x_hbm.at[chunk_slice(k + NBUF)], in_bufs[b], in_sems[b]
                )

        return carry

    lax.fori_loop(0, steps, g_body, 0)

    for b in range(NBUF):
        pltpu.make_async_copy(
            out_bufs[b], out_hbm.at[chunk_slice(jnp.int32(b))], out_sems[b]
        ).wait()


@functools.partial(
    pl.kernel,
    out_type=jax.ShapeDtypeStruct((ROWS, COLS), jnp.float32),
    mesh=plsc.VectorSubcoreMesh(core_axis_name="c", subcore_axis_name="s"),
    scratch_types=[
        [pltpu.VMEM((CHUNK,), jnp.float32)] * NBUF,
        [pltpu.VMEM((CHUNK,), jnp.float32)] * NBUF,
        [pltpu.SemaphoreType.DMA] * NBUF,
        [pltpu.SemaphoreType.DMA] * NBUF,
    ],
)
def _sc_quantize_bottom(x_hbm, out_hbm, in_bufs, out_bufs, in_sems, out_sems):
    _sc_body(x_hbm, out_hbm, in_bufs, out_bufs, in_sems, out_sems)


def _tc_quant_body(x_ref, partial_hbm_ref, o_ref):
    o_ref[...] = _quant_vec(x_ref[...])


def _tc_quantize_top_inplace(x, sc_partial):
    return pl.pallas_call(
        _tc_quant_body,
        grid=(TC_ROWS // TC_BLOCK,),
        in_specs=[
            pl.BlockSpec((TC_BLOCK, COLS), lambda i: (i, 0)),
            pl.BlockSpec(memory_space=pl.ANY),
        ],
        out_specs=pl.BlockSpec((TC_BLOCK, COLS), lambda i: (i, 0)),
        out_shape=jax.ShapeDtypeStruct((ROWS, COLS), jnp.float32),
        input_output_aliases={1: 0},
    )(x, sc_partial)


def kernel(x):
    sc_partial = _sc_quantize_bottom(x)
    return _tc_quantize_top_inplace(x, sc_partial)
